# speedup vs baseline: 1.0037x; 1.0037x over previous
import functools

import jax
import jax.numpy as jnp
from jax import lax
from jax.experimental import pallas as pl
from jax.experimental.pallas import tpu as pltpu
from jax.experimental.pallas import tpu_sc as plsc

N_ROWS = 16384
N_CLASSES = 1000
NC = 2
NS = 16
NW = NC * NS
B_PER_W = N_ROWS // NW
CHUNK = 128
N_CHUNKS = B_PER_W // CHUNK
LANES = 16
SUB = CHUNK // LANES


def _build_kernel():
  mesh = plsc.VectorSubcoreMesh(core_axis_name="c", subcore_axis_name="s")

  @functools.partial(
      pl.kernel,
      mesh=mesh,
      out_type=jax.ShapeDtypeStruct((NW, LANES), jnp.float32),
      scratch_types=[
          pltpu.VMEM((N_CHUNKS, CHUNK), jnp.int32),
          pltpu.VMEM((N_CHUNKS, CHUNK), jnp.float32),
          pltpu.VMEM((LANES,), jnp.float32),
          pltpu.SemaphoreType.DMA,
      ],
  )
  def loss_kernel(flat_hbm, tgt_hbm, out_hbm, idx_v, vals_v, acc_v, sem):
    wid = lax.axis_index("s") * NC + lax.axis_index("c")
    base = wid * B_PER_W
    pltpu.sync_copy(tgt_hbm.at[pl.ds(wid * N_CHUNKS, N_CHUNKS)], idx_v)
    for j in range(N_CHUNKS):
      for k in range(SUB):
        t = idx_v[j, pl.ds(k * LANES, LANES)]
        row = base + j * CHUNK + k * LANES + lax.iota(jnp.int32, LANES)
        idx_v[j, pl.ds(k * LANES, LANES)] = t * N_CLASSES + row
    copies = [
        pltpu.async_copy(flat_hbm.at[idx_v.at[j]], vals_v.at[j], sem)
        for j in range(N_CHUNKS)
    ]
    for c in copies:
      c.wait()
    acc = jnp.zeros((LANES,), jnp.float32)
    for j in range(N_CHUNKS):
      for k in range(SUB):
        acc = acc + vals_v[j, pl.ds(k * LANES, LANES)]
    acc_v[...] = acc
    pltpu.sync_copy(acc_v, out_hbm.at[wid])

  return loss_kernel


_loss_kernel = _build_kernel()


@jax.jit
def kernel(predictions, targets):
  flat = predictions.reshape(-1)
  tgt = targets.astype(jnp.int32).reshape(NW * N_CHUNKS, CHUNK)
  partials = _loss_kernel(flat, tgt)
  return -(partials.sum() / jnp.float32(N_ROWS))


# speedup vs baseline: 4.2786x; 4.2629x over previous
import functools

import jax
import jax.numpy as jnp
from jax import lax
from jax.experimental import pallas as pl
from jax.experimental.pallas import tpu as pltpu
from jax.experimental.pallas import tpu_sc as plsc

N_ROWS = 16384
N_CLASSES = 1000
N_TOUCHED_ROWS = ((N_CLASSES - 1) * N_CLASSES + (N_ROWS - 1)) // N_CLASSES + 1
NC = 2
NS = 16
NW = NC * NS
B_PER_W = N_ROWS // NW
CHUNK = 128
N_CHUNKS = B_PER_W // CHUNK
LANES = 16
SUB = CHUNK // LANES


def _build_kernel():
  mesh = plsc.VectorSubcoreMesh(core_axis_name="c", subcore_axis_name="s")

  @functools.partial(
      pl.kernel,
      mesh=mesh,
      out_type=jax.ShapeDtypeStruct((NW, LANES), jnp.float32),
      scratch_types=[
          pltpu.VMEM((N_CHUNKS, CHUNK), jnp.int32),
          pltpu.VMEM((N_CHUNKS, CHUNK), jnp.float32),
          pltpu.VMEM((LANES,), jnp.float32),
          pltpu.SemaphoreType.DMA,
      ],
  )
  def loss_kernel(flat_hbm, tgt_hbm, out_hbm, idx_v, vals_v, acc_v, sem):
    wid = lax.axis_index("s") * NC + lax.axis_index("c")
    base = wid * B_PER_W
    pltpu.sync_copy(tgt_hbm.at[pl.ds(wid * N_CHUNKS, N_CHUNKS)], idx_v)
    for j in range(N_CHUNKS):
      for k in range(SUB):
        t = idx_v[j, pl.ds(k * LANES, LANES)]
        row = base + j * CHUNK + k * LANES + lax.iota(jnp.int32, LANES)
        idx_v[j, pl.ds(k * LANES, LANES)] = t * N_CLASSES + row
    copies = [
        pltpu.async_copy(flat_hbm.at[idx_v.at[j]], vals_v.at[j], sem)
        for j in range(N_CHUNKS)
    ]
    for c in copies:
      c.wait()
    acc = jnp.zeros((LANES,), jnp.float32)
    for j in range(N_CHUNKS):
      for k in range(SUB):
        acc = acc + vals_v[j, pl.ds(k * LANES, LANES)]
    acc_v[...] = acc
    pltpu.sync_copy(acc_v, out_hbm.at[wid])

  return loss_kernel


_loss_kernel = _build_kernel()


@jax.jit
def kernel(predictions, targets):
  flat = predictions[:N_TOUCHED_ROWS].reshape(-1)
  tgt = targets.astype(jnp.int32).reshape(NW * N_CHUNKS, CHUNK)
  partials = _loss_kernel(flat, tgt)
  return -(partials.sum() / jnp.float32(N_ROWS))
